# baseline (speedup 1.0000x reference)
import functools

import jax
import jax.numpy as jnp
from jax import lax
from jax.experimental import pallas as pl
from jax.experimental.pallas import tpu as pltpu
from jax.experimental.pallas import tpu_sc as plsc

_NC = 2
_NS = 16
_NW = _NC * _NS
_L = 16


def _make_sc_expand(S, D):
    P = 2 * D
    R = S
    slices_per_w = (2 * S) // _NW
    mesh = plsc.VectorSubcoreMesh(core_axis_name="c", subcore_axis_name="s")

    @functools.partial(
        pl.kernel,
        mesh=mesh,
        out_type=jax.ShapeDtypeStruct((2 * S, S * D // P, P), jnp.float32),
        scratch_types=[
            pltpu.VMEM((R, P), jnp.float32),
            pltpu.VMEM((R - 1, P), jnp.float32),
            pltpu.SemaphoreType.DMA,
        ],
    )
    def expand(table_hbm, out_hbm, buf_e, buf_o, sem):
        cid = lax.axis_index("c")
        sid = lax.axis_index("s")
        wid = sid * _NC + cid

        pltpu.sync_copy(table_hbm, buf_e)

        def swap_rows(r, _):
            hi = (R - 1) - r
            for q in range(D // _L):
                for h in (0, D):
                    oh = D - h
                    a = buf_e[r, pl.ds(h + q * _L, _L)]
                    b = buf_e[hi, pl.ds(oh + q * _L, _L)]
                    buf_e[r, pl.ds(h + q * _L, _L)] = b
                    buf_e[hi, pl.ds(oh + q * _L, _L)] = a
            return 0

        lax.fori_loop(0, R // 2, swap_rows, 0)

        def stagger(r, _):
            for q in range(D // _L):
                buf_o[r, pl.ds(q * _L, _L)] = buf_e[r, pl.ds(D + q * _L, _L)]
                buf_o[r, pl.ds(D + q * _L, _L)] = buf_e[r + 1, pl.ds(q * _L, _L)]
            return 0

        lax.fori_loop(0, R - 1, stagger, 0)

        base = wid * slices_per_w
        i0 = lax.rem(base, S)
        copies = []
        for t in range(slices_per_w):
            if t % 2 == 0:
                row0 = (S - i0 - t) // 2
                src = buf_e.at[pl.ds(row0, S // 2)]
            else:
                row0 = ((S - 1) - i0 - t) // 2
                src = buf_o.at[pl.ds(row0, S // 2)]
            copies.append(
                pltpu.async_copy(src, out_hbm.at[base + t], sem)
            )
        for cp in copies:
            cp.wait()

    return expand


def kernel(rel_pos_embedding, batch_size, seq_len):
    n_rows, D = rel_pos_embedding.shape
    S = (n_rows + 1) // 2
    static_batch = 2

    shift = (seq_len - S) + (batch_size - static_batch)
    r = jnp.arange(2 * S, dtype=jnp.int32)
    table_adj = rel_pos_embedding[jnp.clip(r + shift, 0, n_rows - 1)]

    out = _make_sc_expand(S, D)(table_adj.reshape(S, 2 * D))
    return out.reshape(static_batch, S, S, D)


# speedup vs baseline: 1.6768x; 1.6768x over previous
import functools

import jax
import jax.numpy as jnp
from jax import lax
from jax.experimental import pallas as pl
from jax.experimental.pallas import tpu as pltpu
from jax.experimental.pallas import tpu_sc as plsc

_NC = 2
_NS = 16
_NW = _NC * _NS
_L = 16


def _make_sc_expand(S, D):
    rows = 2 * S - 1
    slices_per_w = (2 * S) // _NW
    mesh = plsc.VectorSubcoreMesh(core_axis_name="c", subcore_axis_name="s")

    @functools.partial(
        pl.kernel,
        mesh=mesh,
        out_type=jax.ShapeDtypeStruct((2 * S, S, D), jnp.float32),
        scratch_types=[
            pltpu.VMEM((2 * S, D), jnp.float32),
            pltpu.SemaphoreType.DMA,
        ],
        compiler_params=pltpu.CompilerParams(use_tc_tiling_on_sc=True),
    )
    def expand(table_hbm, out_hbm, buf, sem):
        cid = lax.axis_index("c")
        sid = lax.axis_index("s")
        wid = sid * _NC + cid

        pltpu.sync_copy(table_hbm, buf.at[pl.ds(0, rows)])

        def swap_rows(k, _):
            lo = k
            hi = (rows - 1) - k
            for q in range(D // _L):
                a = buf[lo, pl.ds(q * _L, _L)]
                b = buf[hi, pl.ds(q * _L, _L)]
                buf[lo, pl.ds(q * _L, _L)] = b
                buf[hi, pl.ds(q * _L, _L)] = a
            return 0

        lax.fori_loop(0, (rows - 1) // 2, swap_rows, 0)

        base = wid * slices_per_w
        i0 = lax.rem(base, S)
        copies = []
        for t in range(slices_per_w):
            off = (S - 1) - (i0 + t)
            copies.append(
                pltpu.async_copy(
                    buf.at[pl.ds(off, S)],
                    out_hbm.at[base + t],
                    sem,
                )
            )
        for cp in copies:
            cp.wait()

    return expand


def kernel(rel_pos_embedding, batch_size, seq_len):
    n_rows, D = rel_pos_embedding.shape
    S = (n_rows + 1) // 2
    static_batch = 2

    shift = (seq_len - S) + (batch_size - static_batch)
    r = jnp.arange(n_rows, dtype=jnp.int32)
    table_adj = rel_pos_embedding[jnp.clip(r + shift, 0, n_rows - 1)]

    out = _make_sc_expand(S, D)(table_adj)
    return out.reshape(static_batch, S, S, D)


# speedup vs baseline: 1.6787x; 1.0011x over previous
import functools

import jax
import jax.numpy as jnp
from jax import lax
from jax.experimental import pallas as pl
from jax.experimental.pallas import tpu as pltpu
from jax.experimental.pallas import tpu_sc as plsc

_NC = 2
_NS = 16
_NW = _NC * _NS
_L = 16


def _make_sc_expand(S, D):
    rows = 2 * S - 1
    slices_per_w = (2 * S) // _NW
    mesh = plsc.VectorSubcoreMesh(core_axis_name="c", subcore_axis_name="s")

    @functools.partial(
        pl.kernel,
        mesh=mesh,
        out_type=jax.ShapeDtypeStruct((2 * S, S, D), jnp.float32),
        scratch_types=[
            pltpu.VMEM((2 * S, D), jnp.float32),
            pltpu.SemaphoreType.DMA,
        ],
    )
    def expand(table_hbm, out_hbm, buf, sem):
        cid = lax.axis_index("c")
        sid = lax.axis_index("s")
        wid = sid * _NC + cid

        pltpu.sync_copy(table_hbm, buf.at[pl.ds(0, rows)])

        def swap_rows(k, _):
            lo = k
            hi = (rows - 1) - k
            for q in range(D // _L):
                a = buf[lo, pl.ds(q * _L, _L)]
                b = buf[hi, pl.ds(q * _L, _L)]
                buf[lo, pl.ds(q * _L, _L)] = b
                buf[hi, pl.ds(q * _L, _L)] = a
            return 0

        lax.fori_loop(0, (rows - 1) // 2, swap_rows, 0)

        base = wid * slices_per_w
        i0 = lax.rem(base, S)
        copies = []
        for t in range(slices_per_w):
            off = (S - 1) - (i0 + t)
            copies.append(
                pltpu.async_copy(
                    buf.at[pl.ds(off, S)],
                    out_hbm.at[base + t],
                    sem,
                )
            )
        for cp in copies:
            cp.wait()

    return expand


def kernel(rel_pos_embedding, batch_size, seq_len):
    n_rows, D = rel_pos_embedding.shape
    S = (n_rows + 1) // 2
    static_batch = 2

    shift = (seq_len - S) + (batch_size - static_batch)
    r = jnp.arange(n_rows, dtype=jnp.int32)
    table_adj = rel_pos_embedding[jnp.clip(r + shift, 0, n_rows - 1)]

    out = _make_sc_expand(S, D)(table_adj)
    return out.reshape(static_batch, S, S, D)
